# Initial kernel scaffold; baseline (speedup 1.0000x reference)
import jax
import jax.numpy as jnp
from jax.experimental import pallas as pl


def kernel(x, emb):
    raise NotImplementedError("write your pallas kernel here")


# speedup vs baseline: 1.2757x; 1.2757x over previous
import jax
import jax.numpy as jnp
from jax.experimental import pallas as pl


def _add_kernel(x_ref, emb_ref, o_ref):
    o_ref[...] = x_ref[...] + emb_ref[...]


def kernel(x, emb):
    b, s, d = x.shape
    pe = emb[:s]
    block_s = 512
    grid = (b, s // block_s)
    return pl.pallas_call(
        _add_kernel,
        grid=grid,
        in_specs=[
            pl.BlockSpec((1, block_s, d), lambda i, j: (i, j, 0)),
            pl.BlockSpec((block_s, d), lambda i, j: (j, 0)),
        ],
        out_specs=pl.BlockSpec((1, block_s, d), lambda i, j: (i, j, 0)),
        out_shape=jax.ShapeDtypeStruct((b, s, d), x.dtype),
    )(x, pe)


# speedup vs baseline: 1.4935x; 1.1707x over previous
import jax
import jax.numpy as jnp
from jax.experimental import pallas as pl


def _add_kernel(x_ref, emb_ref, o_ref):
    o_ref[...] = x_ref[...] + emb_ref[...]


def kernel(x, emb):
    b, s, d = x.shape
    pe = emb[:s]
    block_s = 512
    grid = (s // block_s, b)
    return pl.pallas_call(
        _add_kernel,
        grid=grid,
        in_specs=[
            pl.BlockSpec((1, block_s, d), lambda j, i: (i, j, 0)),
            pl.BlockSpec((block_s, d), lambda j, i: (j, 0)),
        ],
        out_specs=pl.BlockSpec((1, block_s, d), lambda j, i: (i, j, 0)),
        out_shape=jax.ShapeDtypeStruct((b, s, d), x.dtype),
    )(x, pe)


# speedup vs baseline: 1.6664x; 1.1158x over previous
import jax
import jax.numpy as jnp
from jax.experimental import pallas as pl


def _add_kernel(x_ref, emb_ref, o_ref):
    o_ref[...] = x_ref[...] + emb_ref[...]


def kernel(x, emb):
    b, s, d = x.shape
    pe = emb[:s]
    block_s = 1024
    grid = (s // block_s, b)
    return pl.pallas_call(
        _add_kernel,
        grid=grid,
        in_specs=[
            pl.BlockSpec((1, block_s, d), lambda j, i: (i, j, 0)),
            pl.BlockSpec((block_s, d), lambda j, i: (j, 0)),
        ],
        out_specs=pl.BlockSpec((1, block_s, d), lambda j, i: (i, j, 0)),
        out_shape=jax.ShapeDtypeStruct((b, s, d), x.dtype),
    )(x, pe)


# speedup vs baseline: 1.7392x; 1.0437x over previous
import jax
import jax.numpy as jnp
from jax.experimental import pallas as pl


def _add_kernel(x_ref, emb_ref, o_ref):
    o_ref[...] = x_ref[...] + emb_ref[...]


def kernel(x, emb):
    b, s, d = x.shape
    pe = emb[:s]
    block_s = 2048
    grid = (s // block_s, b)
    return pl.pallas_call(
        _add_kernel,
        grid=grid,
        in_specs=[
            pl.BlockSpec((1, block_s, d), lambda j, i: (i, j, 0)),
            pl.BlockSpec((block_s, d), lambda j, i: (j, 0)),
        ],
        out_specs=pl.BlockSpec((1, block_s, d), lambda j, i: (i, j, 0)),
        out_shape=jax.ShapeDtypeStruct((b, s, d), x.dtype),
    )(x, pe)
